# baseline (speedup 1.0000x reference)
import functools

import jax
import jax.numpy as jnp
from jax import lax
from jax.experimental import pallas as pl
from jax.experimental.pallas import tpu as pltpu
from jax.experimental.pallas import tpu_sc as plsc

_DIM = 32
_WD = 0.01
_NC = 2
_NS = 16
_NW = _NC * _NS
_LANES = 16
_CHUNK = 128


def _sc_body(nchunks, u_hbm, i_hbm, j_hbm, Wt_hbm, Ht_hbm, x_hbm, reg_hbm,
             idx_u, idx_i, idx_j, u_t, i_t, j_t, x_v, reg_v, sem):
    wid = lax.axis_index("s") * _NC + lax.axis_index("c")
    bpw = nchunks * _CHUNK
    base = wid * bpw

    cps = [
        pltpu.async_copy(u_hbm.at[pl.ds(base, bpw)], idx_u, sem),
        pltpu.async_copy(i_hbm.at[pl.ds(base, bpw)], idx_i, sem),
        pltpu.async_copy(j_hbm.at[pl.ds(base, bpw)], idx_j, sem),
    ]
    for cp in cps:
        cp.wait()

    def gather_d(d):
        cps = []
        for k in range(nchunks):
            isl = pl.ds(k * _CHUNK, _CHUNK)
            cps.append(pltpu.async_copy(
                Wt_hbm.at[d].at[idx_u.at[pl.ds(k * _CHUNK, _CHUNK)]],
                u_t.at[d, isl], sem))
            cps.append(pltpu.async_copy(
                Ht_hbm.at[d].at[idx_i.at[pl.ds(k * _CHUNK, _CHUNK)]],
                i_t.at[d, isl], sem))
            cps.append(pltpu.async_copy(
                Ht_hbm.at[d].at[idx_j.at[pl.ds(k * _CHUNK, _CHUNK)]],
                j_t.at[d, isl], sem))
        for cp in cps:
            cp.wait()

    pl.loop(0, _DIM)(gather_d)

    def group(g, reg_acc):
        sl = pl.ds(g * _LANES, _LANES)
        acc_ui = jnp.zeros((_LANES,), jnp.float32)
        acc_uj = jnp.zeros((_LANES,), jnp.float32)
        sq = reg_acc
        for d in range(_DIM):
            uc = u_t[d, sl]
            ic = i_t[d, sl]
            jc = j_t[d, sl]
            acc_ui = acc_ui + uc * ic
            acc_uj = acc_uj + uc * jc
            sq = sq + (uc * uc + ic * ic + jc * jc)
        x_v[sl] = acc_ui - acc_uj
        return sq

    ngroups = bpw // _LANES
    reg = lax.fori_loop(0, ngroups, group, jnp.zeros((_LANES,), jnp.float32))
    reg_v[...] = reg
    pltpu.sync_copy(x_v, x_hbm.at[pl.ds(base, bpw)])
    pltpu.sync_copy(reg_v, reg_hbm.at[wid])


def _tc_body(x_ref, reg_ref, out_ref):
    xs = x_ref[...]
    ls = jnp.minimum(xs, 0.0) - jnp.log1p(jnp.exp(-jnp.abs(xs)))
    out_ref[0, 0] = -jnp.sum(ls) + _WD * jnp.sum(reg_ref[...])


def kernel(u, i, j, W, H):
    B = u.shape[0]
    nchunks = B // (_NW * _CHUNK)
    bpw = nchunks * _CHUNK
    mesh = plsc.VectorSubcoreMesh(core_axis_name="c", subcore_axis_name="s")

    sc = pl.kernel(
        functools.partial(_sc_body, nchunks),
        out_type=(
            jax.ShapeDtypeStruct((B,), jnp.float32),
            jax.ShapeDtypeStruct((_NW, _LANES), jnp.float32),
        ),
        mesh=mesh,
        compiler_params=pltpu.CompilerParams(
            needs_layout_passes=False, use_tc_tiling_on_sc=False),
        scratch_types=[
            pltpu.VMEM((bpw,), jnp.int32),
            pltpu.VMEM((bpw,), jnp.int32),
            pltpu.VMEM((bpw,), jnp.int32),
            pltpu.VMEM((_DIM, bpw), jnp.float32),
            pltpu.VMEM((_DIM, bpw), jnp.float32),
            pltpu.VMEM((_DIM, bpw), jnp.float32),
            pltpu.VMEM((bpw,), jnp.float32),
            pltpu.VMEM((_LANES,), jnp.float32),
            pltpu.SemaphoreType.DMA,
        ],
    )

    x, reg = sc(u.astype(jnp.int32), i.astype(jnp.int32), j.astype(jnp.int32),
                W.T, H.T)

    out = pl.pallas_call(
        _tc_body,
        out_shape=jax.ShapeDtypeStruct((1, 1), jnp.float32),
        out_specs=pl.BlockSpec(memory_space=pltpu.SMEM),
    )(x.reshape(B // 128, 128), reg)
    return out.reshape(())


# speedup vs baseline: 1.0027x; 1.0027x over previous
import functools

import jax
import jax.numpy as jnp
from jax import lax
from jax.experimental import pallas as pl
from jax.experimental.pallas import tpu as pltpu
from jax.experimental.pallas import tpu_sc as plsc

_DIM = 32
_WD = 0.01
_NC = 2
_NS = 16
_NW = _NC * _NS
_LANES = 16
_CHUNK = 128


def _sc_body(nchunks, u_hbm, i_hbm, j_hbm, Wt_hbm, Ht_hbm, x_hbm, reg_hbm,
             idx_u, idx_i, idx_j, u_t, i_t, j_t, x_v, reg_v, sem):
    wid = lax.axis_index("s") * _NC + lax.axis_index("c")
    bpw = nchunks * _CHUNK
    base = wid * bpw

    cps = [
        pltpu.async_copy(u_hbm.at[pl.ds(base, bpw)], idx_u, sem),
        pltpu.async_copy(i_hbm.at[pl.ds(base, bpw)], idx_i, sem),
        pltpu.async_copy(j_hbm.at[pl.ds(base, bpw)], idx_j, sem),
    ]
    for cp in cps:
        cp.wait()

    def gather_d(d):
        cps = []
        for k in range(nchunks):
            isl = pl.ds(k * _CHUNK, _CHUNK)
            cps.append(pltpu.async_copy(
                Wt_hbm.at[d].at[idx_u.at[pl.ds(k * _CHUNK, _CHUNK)]],
                u_t.at[d, isl], sem))
            cps.append(pltpu.async_copy(
                Ht_hbm.at[d].at[idx_i.at[pl.ds(k * _CHUNK, _CHUNK)]],
                i_t.at[d, isl], sem))
            cps.append(pltpu.async_copy(
                Ht_hbm.at[d].at[idx_j.at[pl.ds(k * _CHUNK, _CHUNK)]],
                j_t.at[d, isl], sem))
        for cp in cps:
            cp.wait()

    pl.loop(0, _DIM)(gather_d)

    def group(g, reg_acc):
        sl = pl.ds(g * _LANES, _LANES)
        acc_ui = jnp.zeros((_LANES,), jnp.float32)
        acc_uj = jnp.zeros((_LANES,), jnp.float32)
        sq = reg_acc
        for d in range(_DIM):
            uc = u_t[d, sl]
            ic = i_t[d, sl]
            jc = j_t[d, sl]
            acc_ui = acc_ui + uc * ic
            acc_uj = acc_uj + uc * jc
            sq = sq + (uc * uc + ic * ic + jc * jc)
        x_v[sl] = acc_ui - acc_uj
        return sq

    ngroups = bpw // _LANES
    reg = lax.fori_loop(0, ngroups, group, jnp.zeros((_LANES,), jnp.float32))
    reg_v[...] = reg
    pltpu.sync_copy(x_v, x_hbm.at[pl.ds(base, bpw)])
    pltpu.sync_copy(reg_v, reg_hbm.at[wid])


def _tc_body(x_ref, reg_ref, out_ref):
    xs = x_ref[...]
    ls = jnp.minimum(xs, 0.0) - jnp.log1p(jnp.exp(-jnp.abs(xs)))
    out_ref[0, 0] = -jnp.sum(ls) + _WD * jnp.sum(reg_ref[...])


def kernel(u, i, j, W, H):
    B = u.shape[0]
    nchunks = B // (_NW * _CHUNK)
    bpw = nchunks * _CHUNK
    mesh = plsc.VectorSubcoreMesh(core_axis_name="c", subcore_axis_name="s")

    sc = pl.kernel(
        functools.partial(_sc_body, nchunks),
        out_type=(
            jax.ShapeDtypeStruct((B,), jnp.float32),
            jax.ShapeDtypeStruct((_NW, _LANES), jnp.float32),
        ),
        mesh=mesh,
        compiler_params=pltpu.CompilerParams(
            needs_layout_passes=False, use_tc_tiling_on_sc=False),
        scratch_types=[
            pltpu.VMEM((bpw,), jnp.int32),
            pltpu.VMEM((bpw,), jnp.int32),
            pltpu.VMEM((bpw,), jnp.int32),
            pltpu.VMEM((_DIM, bpw), jnp.float32),
            pltpu.VMEM((_DIM, bpw), jnp.float32),
            pltpu.VMEM((_DIM, bpw), jnp.float32),
            pltpu.VMEM((bpw,), jnp.float32),
            pltpu.VMEM((_LANES,), jnp.float32),
            pltpu.SemaphoreType.DMA,
        ],
    )

    Wt, Ht = jax.lax.optimization_barrier((W.T, H.T))
    x, reg = sc(u.astype(jnp.int32), i.astype(jnp.int32), j.astype(jnp.int32),
                Wt, Ht)

    out = pl.pallas_call(
        _tc_body,
        out_shape=jax.ShapeDtypeStruct((1, 1), jnp.float32),
        out_specs=pl.BlockSpec(memory_space=pltpu.SMEM),
    )(x.reshape(B // 128, 128), reg)
    return out.reshape(())


# speedup vs baseline: 5.5868x; 5.5716x over previous
import functools

import jax
import jax.numpy as jnp
from jax import lax
from jax.experimental import pallas as pl
from jax.experimental.pallas import tpu as pltpu
from jax.experimental.pallas import tpu_sc as plsc

_DIM = 32
_WD = 0.01
_NC = 2
_NS = 16
_NW = _NC * _NS
_LANES = 16
_CHUNK = 128
_GROUP = 128 // _DIM


def _detile_body(in_ref, out_ref):
    x = in_ref[...]
    c = x.shape[1]
    out_ref[...] = x.T.reshape(c // _GROUP, 128)


def _detile(xt, cols):
    v = xt.shape[1]
    grid = (v + cols - 1) // cols
    return pl.pallas_call(
        _detile_body,
        grid=(grid,),
        in_specs=[pl.BlockSpec((_DIM, cols), lambda t: (0, t))],
        out_specs=pl.BlockSpec((cols // _GROUP, 128), lambda t: (t, 0)),
        out_shape=jax.ShapeDtypeStruct((v // _GROUP, 128), jnp.float32),
    )(xt)


def _sc_body(nchunks, u_hbm, i_hbm, j_hbm, W4_hbm, H4_hbm, x_hbm, reg_hbm,
             idx_u, idx_i, idx_j, g_u, g_i, g_j, u_r, i_r, j_r,
             x_v, reg_v, sem):
    wid = lax.axis_index("s") * _NC + lax.axis_index("c")
    bpw = nchunks * _CHUNK
    base = wid * bpw

    cps = []
    for k in range(nchunks):
        sl = pl.ds(base + k * _CHUNK, _CHUNK)
        cps.append(pltpu.async_copy(u_hbm.at[sl], idx_u.at[k], sem))
        cps.append(pltpu.async_copy(i_hbm.at[sl], idx_i.at[k], sem))
        cps.append(pltpu.async_copy(j_hbm.at[sl], idx_j.at[k], sem))
    for cp in cps:
        cp.wait()

    for k in range(nchunks):
        for o in range(_CHUNK // _LANES):
            sl = pl.ds(o * _LANES, _LANES)
            g_u[k, sl] = lax.shift_right_logical(idx_u[k, sl], 2)
            g_i[k, sl] = lax.shift_right_logical(idx_i[k, sl], 2)
            g_j[k, sl] = lax.shift_right_logical(idx_j[k, sl], 2)

    lane = lax.broadcasted_iota(jnp.int32, (_LANES,), 0)

    def pass_body(p, sq0):
        cps = [
            pltpu.async_copy(W4_hbm.at[g_u.at[p]], u_r, sem),
            pltpu.async_copy(H4_hbm.at[g_i.at[p]], i_r, sem),
            pltpu.async_copy(H4_hbm.at[g_j.at[p]], j_r, sem),
        ]
        for cp in cps:
            cp.wait()

        def group(gg, sq):
            sl = pl.ds(gg * _LANES, _LANES)
            rowi = lane + gg * _LANES
            cb_u = (idx_u[p, sl] & (_GROUP - 1)) * _DIM
            cb_i = (idx_i[p, sl] & (_GROUP - 1)) * _DIM
            cb_j = (idx_j[p, sl] & (_GROUP - 1)) * _DIM
            acc_ui = jnp.zeros((_LANES,), jnp.float32)
            acc_uj = jnp.zeros((_LANES,), jnp.float32)
            for d in range(_DIM):
                uc = plsc.load_gather(u_r, [rowi, cb_u + d])
                ic = plsc.load_gather(i_r, [rowi, cb_i + d])
                jc = plsc.load_gather(j_r, [rowi, cb_j + d])
                acc_ui = acc_ui + uc * ic
                acc_uj = acc_uj + uc * jc
                sq = sq + (uc * uc + ic * ic + jc * jc)
            x_v[pl.ds(p * _CHUNK + gg * _LANES, _LANES)] = acc_ui - acc_uj
            return sq

        return lax.fori_loop(0, _CHUNK // _LANES, group, sq0)

    reg = lax.fori_loop(0, nchunks, pass_body,
                        jnp.zeros((_LANES,), jnp.float32))
    reg_v[...] = reg
    pltpu.sync_copy(x_v, x_hbm.at[pl.ds(base, bpw)])
    pltpu.sync_copy(reg_v, reg_hbm.at[wid])


def _tc_body(x_ref, reg_ref, out_ref):
    xs = x_ref[...]
    ls = jnp.minimum(xs, 0.0) - jnp.log1p(jnp.exp(-jnp.abs(xs)))
    out_ref[0, 0] = -jnp.sum(ls) + _WD * jnp.sum(reg_ref[...])


def kernel(u, i, j, W, H):
    B = u.shape[0]
    nchunks = B // (_NW * _CHUNK)
    bpw = nchunks * _CHUNK
    mesh = plsc.VectorSubcoreMesh(core_axis_name="c", subcore_axis_name="s")

    V = W.shape[0]
    W4 = jnp.reshape(W, (V // _GROUP, 128))
    H4 = jnp.reshape(H, (V // _GROUP, 128))

    sc = pl.kernel(
        functools.partial(_sc_body, nchunks),
        out_type=(
            jax.ShapeDtypeStruct((B,), jnp.float32),
            jax.ShapeDtypeStruct((_NW, _LANES), jnp.float32),
        ),
        mesh=mesh,
        compiler_params=pltpu.CompilerParams(
            needs_layout_passes=False, use_tc_tiling_on_sc=False),
        scratch_types=[
            pltpu.VMEM((nchunks, _CHUNK), jnp.int32),
            pltpu.VMEM((nchunks, _CHUNK), jnp.int32),
            pltpu.VMEM((nchunks, _CHUNK), jnp.int32),
            pltpu.VMEM((nchunks, _CHUNK), jnp.int32),
            pltpu.VMEM((nchunks, _CHUNK), jnp.int32),
            pltpu.VMEM((nchunks, _CHUNK), jnp.int32),
            pltpu.VMEM((_CHUNK, 128), jnp.float32),
            pltpu.VMEM((_CHUNK, 128), jnp.float32),
            pltpu.VMEM((_CHUNK, 128), jnp.float32),
            pltpu.VMEM((bpw,), jnp.float32),
            pltpu.VMEM((_LANES,), jnp.float32),
            pltpu.SemaphoreType.DMA,
        ],
    )

    x, reg = sc(u.astype(jnp.int32), i.astype(jnp.int32), j.astype(jnp.int32),
                W4, H4)

    out = pl.pallas_call(
        _tc_body,
        out_shape=jax.ShapeDtypeStruct((1, 1), jnp.float32),
        out_specs=pl.BlockSpec(memory_space=pltpu.SMEM),
    )(x.reshape(B // 128, 128), reg)
    return out.reshape(())


# speedup vs baseline: 5.5931x; 1.0011x over previous
import functools

import jax
import jax.numpy as jnp
from jax import lax
from jax.experimental import pallas as pl
from jax.experimental.pallas import tpu as pltpu
from jax.experimental.pallas import tpu_sc as plsc

_DIM = 32
_WD = 0.01
_NC = 2
_NS = 16
_NW = _NC * _NS
_LANES = 16
_CHUNK = 128
_GROUP = 128 // _DIM


def _sc_body(nchunks, u_hbm, i_hbm, j_hbm, W4_hbm, H4_hbm, x_hbm, reg_hbm,
             idx_u, idx_i, idx_j, g_u, g_i, g_j, u_r, i_r, j_r,
             x_v, reg_v, sem):
    wid = lax.axis_index("s") * _NC + lax.axis_index("c")
    bpw = nchunks * _CHUNK
    base = wid * bpw

    cps = []
    for k in range(nchunks):
        sl = pl.ds(base + k * _CHUNK, _CHUNK)
        cps.append(pltpu.async_copy(u_hbm.at[sl], idx_u.at[k], sem))
        cps.append(pltpu.async_copy(i_hbm.at[sl], idx_i.at[k], sem))
        cps.append(pltpu.async_copy(j_hbm.at[sl], idx_j.at[k], sem))
    for cp in cps:
        cp.wait()

    for k in range(nchunks):
        for o in range(_CHUNK // _LANES):
            sl = pl.ds(o * _LANES, _LANES)
            g_u[k, sl] = lax.shift_right_logical(idx_u[k, sl], 2)
            g_i[k, sl] = lax.shift_right_logical(idx_i[k, sl], 2)
            g_j[k, sl] = lax.shift_right_logical(idx_j[k, sl], 2)

    lane = lax.broadcasted_iota(jnp.int32, (_LANES,), 0)

    def pass_body(p, sq0):
        cps = [
            pltpu.async_copy(W4_hbm.at[g_u.at[p]], u_r, sem),
            pltpu.async_copy(H4_hbm.at[g_i.at[p]], i_r, sem),
            pltpu.async_copy(H4_hbm.at[g_j.at[p]], j_r, sem),
        ]
        for cp in cps:
            cp.wait()

        def group(gg, sq):
            sl = pl.ds(gg * _LANES, _LANES)
            rowi = lane + gg * _LANES
            cb_u = (idx_u[p, sl] & (_GROUP - 1)) * _DIM
            cb_i = (idx_i[p, sl] & (_GROUP - 1)) * _DIM
            cb_j = (idx_j[p, sl] & (_GROUP - 1)) * _DIM
            acc_ui = jnp.zeros((_LANES,), jnp.float32)
            acc_uj = jnp.zeros((_LANES,), jnp.float32)
            for d in range(_DIM):
                uc = plsc.load_gather(u_r, [rowi, cb_u + d])
                ic = plsc.load_gather(i_r, [rowi, cb_i + d])
                jc = plsc.load_gather(j_r, [rowi, cb_j + d])
                acc_ui = acc_ui + uc * ic
                acc_uj = acc_uj + uc * jc
                sq = sq + (uc * uc + ic * ic + jc * jc)
            x_v[pl.ds(p * _CHUNK + gg * _LANES, _LANES)] = acc_ui - acc_uj
            return sq

        return lax.fori_loop(0, _CHUNK // _LANES, group, sq0)

    reg = lax.fori_loop(0, nchunks, pass_body,
                        jnp.zeros((_LANES,), jnp.float32))
    reg_v[...] = reg
    pltpu.sync_copy(x_v, x_hbm.at[pl.ds(base, bpw)])
    pltpu.sync_copy(reg_v, reg_hbm.at[wid])


def _tc_body(x_ref, reg_ref, out_ref):
    xs = x_ref[...]
    ls = jnp.minimum(xs, 0.0) - jnp.log1p(jnp.exp(-jnp.abs(xs)))
    out_ref[0, 0] = -jnp.sum(ls) + _WD * jnp.sum(reg_ref[...])


def kernel(u, i, j, W, H):
    B = u.shape[0]
    nchunks = B // (_NW * _CHUNK)
    bpw = nchunks * _CHUNK
    mesh = plsc.VectorSubcoreMesh(core_axis_name="c", subcore_axis_name="s")

    V = W.shape[0]
    W4 = jnp.reshape(W, (V // _GROUP, 128))
    H4 = jnp.reshape(H, (V // _GROUP, 128))

    sc = pl.kernel(
        functools.partial(_sc_body, nchunks),
        out_type=(
            jax.ShapeDtypeStruct((B,), jnp.float32),
            jax.ShapeDtypeStruct((_NW, _LANES), jnp.float32),
        ),
        mesh=mesh,
        compiler_params=pltpu.CompilerParams(
            needs_layout_passes=False, use_tc_tiling_on_sc=False),
        scratch_types=[
            pltpu.VMEM((nchunks, _CHUNK), jnp.int32),
            pltpu.VMEM((nchunks, _CHUNK), jnp.int32),
            pltpu.VMEM((nchunks, _CHUNK), jnp.int32),
            pltpu.VMEM((nchunks, _CHUNK), jnp.int32),
            pltpu.VMEM((nchunks, _CHUNK), jnp.int32),
            pltpu.VMEM((nchunks, _CHUNK), jnp.int32),
            pltpu.VMEM((_CHUNK, 128), jnp.float32),
            pltpu.VMEM((_CHUNK, 128), jnp.float32),
            pltpu.VMEM((_CHUNK, 128), jnp.float32),
            pltpu.VMEM((bpw,), jnp.float32),
            pltpu.VMEM((_LANES,), jnp.float32),
            pltpu.SemaphoreType.DMA,
        ],
    )

    x, reg = sc(u.astype(jnp.int32), i.astype(jnp.int32), j.astype(jnp.int32),
                W4, H4)

    out = pl.pallas_call(
        _tc_body,
        out_shape=jax.ShapeDtypeStruct((1, 1), jnp.float32),
        out_specs=pl.BlockSpec(memory_space=pltpu.SMEM),
    )(x.reshape(B // 128, 128), reg)
    return out.reshape(())


# speedup vs baseline: 5.6241x; 1.0055x over previous
import functools

import jax
import jax.numpy as jnp
from jax import lax
from jax.experimental import pallas as pl
from jax.experimental.pallas import tpu as pltpu
from jax.experimental.pallas import tpu_sc as plsc

_DIM = 32
_WD = 0.01
_NC = 2
_NS = 16
_NW = _NC * _NS
_LANES = 16
_CHUNK = 128


def _sc_body(nchunks, u_hbm, i_hbm, j_hbm, W_hbm, H_hbm, x_hbm, reg_hbm,
             idx_u, idx_i, idx_j, u_r, i_r, j_r, x_v, reg_v, sem):
    wid = lax.axis_index("s") * _NC + lax.axis_index("c")
    bpw = nchunks * _CHUNK
    base = wid * bpw

    cps = []
    for k in range(nchunks):
        sl = pl.ds(base + k * _CHUNK, _CHUNK)
        cps.append(pltpu.async_copy(u_hbm.at[sl], idx_u.at[k], sem))
        cps.append(pltpu.async_copy(i_hbm.at[sl], idx_i.at[k], sem))
        cps.append(pltpu.async_copy(j_hbm.at[sl], idx_j.at[k], sem))
    for cp in cps:
        cp.wait()

    lane = lax.broadcasted_iota(jnp.int32, (_LANES,), 0)

    def pass_body(p, sq0):
        cps = [
            pltpu.async_copy(W_hbm.at[idx_u.at[p]], u_r, sem),
            pltpu.async_copy(H_hbm.at[idx_i.at[p]], i_r, sem),
            pltpu.async_copy(H_hbm.at[idx_j.at[p]], j_r, sem),
        ]
        for cp in cps:
            cp.wait()

        def group(gg, sq):
            rowi = lane + gg * _LANES
            acc_ui = jnp.zeros((_LANES,), jnp.float32)
            acc_uj = jnp.zeros((_LANES,), jnp.float32)
            for d in range(_DIM):
                col = jnp.full((_LANES,), d, jnp.int32)
                uc = plsc.load_gather(u_r, [rowi, col])
                ic = plsc.load_gather(i_r, [rowi, col])
                jc = plsc.load_gather(j_r, [rowi, col])
                acc_ui = acc_ui + uc * ic
                acc_uj = acc_uj + uc * jc
                sq = sq + (uc * uc + ic * ic + jc * jc)
            x_v[pl.ds(p * _CHUNK + gg * _LANES, _LANES)] = acc_ui - acc_uj
            return sq

        return lax.fori_loop(0, _CHUNK // _LANES, group, sq0)

    reg = lax.fori_loop(0, nchunks, pass_body,
                        jnp.zeros((_LANES,), jnp.float32))
    reg_v[...] = reg
    pltpu.sync_copy(x_v, x_hbm.at[pl.ds(base, bpw)])
    pltpu.sync_copy(reg_v, reg_hbm.at[wid])


def _tc_body(x_ref, reg_ref, out_ref):
    xs = x_ref[...]
    ls = jnp.minimum(xs, 0.0) - jnp.log1p(jnp.exp(-jnp.abs(xs)))
    out_ref[0, 0] = -jnp.sum(ls) + _WD * jnp.sum(reg_ref[...])


def kernel(u, i, j, W, H):
    B = u.shape[0]
    nchunks = B // (_NW * _CHUNK)
    bpw = nchunks * _CHUNK
    mesh = plsc.VectorSubcoreMesh(core_axis_name="c", subcore_axis_name="s")

    sc = pl.kernel(
        functools.partial(_sc_body, nchunks),
        out_type=(
            jax.ShapeDtypeStruct((B,), jnp.float32),
            jax.ShapeDtypeStruct((_NW, _LANES), jnp.float32),
        ),
        mesh=mesh,
        compiler_params=pltpu.CompilerParams(
            needs_layout_passes=False, use_tc_tiling_on_sc=False),
        scratch_types=[
            pltpu.VMEM((nchunks, _CHUNK), jnp.int32),
            pltpu.VMEM((nchunks, _CHUNK), jnp.int32),
            pltpu.VMEM((nchunks, _CHUNK), jnp.int32),
            pltpu.VMEM((_CHUNK, _DIM), jnp.float32),
            pltpu.VMEM((_CHUNK, _DIM), jnp.float32),
            pltpu.VMEM((_CHUNK, _DIM), jnp.float32),
            pltpu.VMEM((bpw,), jnp.float32),
            pltpu.VMEM((_LANES,), jnp.float32),
            pltpu.SemaphoreType.DMA,
        ],
    )

    x, reg = sc(u.astype(jnp.int32), i.astype(jnp.int32), j.astype(jnp.int32),
                W, H)

    out = pl.pallas_call(
        _tc_body,
        out_shape=jax.ShapeDtypeStruct((1, 1), jnp.float32),
        out_specs=pl.BlockSpec(memory_space=pltpu.SMEM),
    )(x.reshape(B // 128, 128), reg)
    return out.reshape(())
